# Initial kernel scaffold; baseline (speedup 1.0000x reference)
import jax
import jax.numpy as jnp
from jax.experimental import pallas as pl


def kernel(rays_points_world, rays_directions_world, vertices_posed, Ts):
    raise NotImplementedError("write your pallas kernel here")


# speedup vs baseline: 16.7526x; 16.7526x over previous
import functools

import jax
import jax.numpy as jnp
from jax import lax
from jax.experimental import pallas as pl
from jax.experimental.pallas import tpu as pltpu
from jax.experimental.pallas import tpu_sc as plsc

_R, _P, _V = 512, 32, 6890
_N = _R * _P
_VPAD = 7168
_BN = 512
_BV = 1024
_NVT = _VPAD // _BV
_VPAD2 = 6912
_BD = 512

_NC, _NS = 2, 16
_NW = _NC * _NS
_BPW = _N // _NW


def _argmin_body(pts_ref, verts_ref, idx_ref):
    pts = pts_ref[...]
    x2 = ((pts[:, 0:1] * pts[:, 0:1] + pts[:, 1:2] * pts[:, 1:2])
          + pts[:, 2:3] * pts[:, 2:3])
    big = jnp.float32(jnp.inf)
    acc_f = jnp.full((_BN, 1), big, jnp.float32)
    acc_i = jnp.zeros((_BN, 1), jnp.int32)
    lane = lax.broadcasted_iota(jnp.int32, (_BN, _BV), 1)
    pts_b = pts.astype(jnp.bfloat16)
    for i in range(_NVT):
        vt = verts_ref[:, i * _BV:(i + 1) * _BV]
        y2 = ((vt[0:1, :] * vt[0:1, :] + vt[1:2, :] * vt[1:2, :])
              + vt[2:3, :] * vt[2:3, :])
        xy = jnp.dot(pts_b, vt.astype(jnp.bfloat16),
                     preferred_element_type=jnp.float32)
        s = jnp.sqrt(jnp.maximum(x2 + y2 - 2.0 * xy, 0.0))
        gidx = i * _BV + lane
        s = jnp.where(gidx < _V, s, big)
        cm = jnp.min(s, axis=1, keepdims=True)
        ci = jnp.min(jnp.where(s == cm, gidx, jnp.int32(2**31 - 1)),
                     axis=1, keepdims=True)
        acc_i = jnp.where(cm < acc_f, ci, acc_i)
        acc_f = jnp.where(acc_f < cm, acc_f,
                          cm.astype(jnp.bfloat16).astype(jnp.float32))
    idx_ref[0, 0, :] = acc_i[:, 0]


def _nearest_idx(pts8, verts_t):
    out = pl.pallas_call(
        _argmin_body,
        grid=(_N // _BN,),
        in_specs=[
            pl.BlockSpec((_BN, 8), lambda n: (n, 0)),
            pl.BlockSpec((8, _VPAD), lambda n: (0, 0)),
        ],
        out_specs=pl.BlockSpec((1, 1, _BN), lambda n: (n, 0, 0)),
        out_shape=jax.ShapeDtypeStruct((_N // _BN, 1, _BN), jnp.int32),
    )(pts8, verts_t)
    return out.reshape(_N)


def _inv_body(t_ref, o_ref):
    a = [t_ref[i] for i in range(16)]
    (a00, a01, a02, a03,
     a10, a11, a12, a13,
     a20, a21, a22, a23,
     a30, a31, a32, a33) = a
    s0 = a00 * a11 - a10 * a01
    s1 = a00 * a12 - a10 * a02
    s2 = a00 * a13 - a10 * a03
    s3 = a01 * a12 - a11 * a02
    s4 = a01 * a13 - a11 * a03
    s5 = a02 * a13 - a12 * a03
    c5 = a22 * a33 - a32 * a23
    c4 = a21 * a33 - a31 * a23
    c3 = a21 * a32 - a31 * a22
    c2 = a20 * a33 - a30 * a23
    c1 = a20 * a32 - a30 * a22
    c0 = a20 * a31 - a30 * a21
    det = s0 * c5 - s1 * c4 + s2 * c3 + s3 * c2 - s4 * c1 + s5 * c0
    inv_det = 1.0 / det
    b = [
        (a11 * c5 - a12 * c4 + a13 * c3) * inv_det,
        (-a01 * c5 + a02 * c4 - a03 * c3) * inv_det,
        (a31 * s5 - a32 * s4 + a33 * s3) * inv_det,
        (-a21 * s5 + a22 * s4 - a23 * s3) * inv_det,
        (-a10 * c5 + a12 * c2 - a13 * c1) * inv_det,
        (a00 * c5 - a02 * c2 + a03 * c1) * inv_det,
        (-a30 * s5 + a32 * s2 - a33 * s1) * inv_det,
        (a20 * s5 - a22 * s2 + a23 * s1) * inv_det,
        (a10 * c4 - a11 * c2 + a13 * c0) * inv_det,
        (-a00 * c4 + a01 * c2 - a03 * c0) * inv_det,
        (a30 * s4 - a31 * s2 + a33 * s0) * inv_det,
        (-a20 * s4 + a21 * s2 - a23 * s0) * inv_det,
        (-a10 * c3 + a11 * c1 - a12 * c0) * inv_det,
        (a00 * c3 - a01 * c1 + a02 * c0) * inv_det,
        (-a30 * s3 + a31 * s1 - a32 * s0) * inv_det,
        (a20 * s3 - a21 * s1 + a22 * s0) * inv_det,
    ]
    for i in range(16):
        o_ref[i] = b[i]


def _invert_all(t_rows):
    return pl.pallas_call(
        _inv_body,
        out_shape=jax.ShapeDtypeStruct((16, _VPAD2 // 128, 128), jnp.float32),
    )(t_rows)


_GCH = 128
_NCHUNK = _BPW // _GCH


def _sc_gather(table, idx3):
    mesh = plsc.VectorSubcoreMesh(core_axis_name="c", subcore_axis_name="s")

    @functools.partial(
        pl.kernel,
        mesh=mesh,
        out_type=jax.ShapeDtypeStruct((_NW, _BPW, 128), jnp.float32),
        scratch_types=[
            pltpu.VMEM((_NCHUNK, _GCH), jnp.int32),
            pltpu.VMEM((_BPW, 128), jnp.float32),
            pltpu.SemaphoreType.DMA,
        ],
    )
    def gk(table_hbm, idx_hbm, out_hbm, idx_v, rows_v, sem):
        wid = lax.axis_index("s") * _NC + lax.axis_index("c")
        pltpu.sync_copy(idx_hbm.at[wid], idx_v)
        copies = [
            pltpu.async_copy(table_hbm.at[idx_v.at[k]],
                             rows_v.at[pl.ds(k * _GCH, _GCH)], sem)
            for k in range(_NCHUNK)
        ]
        for c in copies:
            c.wait()
        pltpu.sync_copy(rows_v, out_hbm.at[wid])

    return gk(table, idx3)


def _apply_body(pts_ref, tg_ref, op_ref, od_ref):
    pts = pts_ref[...]
    wr = lax.broadcasted_iota(jnp.int32, (8, 16), 0)
    wk = lax.broadcasted_iota(jnp.int32, (8, 16), 1) % 4
    w = jnp.where((wr == wk) & (wk < 3),
                  jnp.where(wk < 2, -1.0, 1.0), 0.0).astype(jnp.float32)
    ek = lax.broadcasted_iota(jnp.int32, (1, 16), 1)
    e = jnp.where(ek % 4 == 3, 1.0, 0.0).astype(jnp.float32)
    sk = lax.broadcasted_iota(jnp.int32, (16, 4), 0)
    si = lax.broadcasted_iota(jnp.int32, (16, 4), 1)
    s = jnp.where(sk // 4 == si, 1.0, 0.0).astype(jnp.float32)
    h_rep = jnp.dot(pts, w, preferred_element_type=jnp.float32,
                    precision=lax.Precision.HIGHEST) + e
    prod = tg_ref[:, :16] * h_rep
    can = jnp.dot(prod, s, preferred_element_type=jnp.float32,
                  precision=lax.Precision.HIGHEST)
    nxt = jnp.roll(can, -1, axis=0)
    prv = jnp.roll(can, 1, axis=0)
    rowi = lax.broadcasted_iota(jnp.int32, (_BD, 4), 0)
    is_last = (rowi % _P) == (_P - 1)
    d = jnp.where(is_last, can - prv, nxt - can)
    lanei = lax.broadcasted_iota(jnp.int32, (_BD, 4), 1)
    d = jnp.where(lanei < 3, d, 0.0)
    n2 = jnp.sum(d * d, axis=1, keepdims=True)
    nrm = jnp.maximum(jnp.sqrt(n2), 1e-12)
    dd = d / nrm
    fl = lax.broadcasted_iota(jnp.int32, (1, 4), 1)
    flip = jnp.where(fl < 2, -1.0, 1.0).astype(jnp.float32)
    op_ref[...] = can * flip
    od_ref[...] = dd * flip


def _apply_all(pts8, tg):
    return pl.pallas_call(
        _apply_body,
        grid=(_N // _BD,),
        in_specs=[
            pl.BlockSpec((_BD, 8), lambda n: (n, 0)),
            pl.BlockSpec((_BD, 128), lambda n: (n, 0)),
        ],
        out_specs=[
            pl.BlockSpec((_BD, 4), lambda n: (n, 0)),
            pl.BlockSpec((_BD, 4), lambda n: (n, 0)),
        ],
        out_shape=[
            jax.ShapeDtypeStruct((_N, 4), jnp.float32),
            jax.ShapeDtypeStruct((_N, 4), jnp.float32),
        ],
    )(pts8, tg)


def kernel(rays_points_world, rays_directions_world, vertices_posed, Ts):
    del rays_directions_world
    pts = rays_points_world.reshape(_N, 3)
    pts8 = jnp.zeros((_N, 8), jnp.float32).at[:, :3].set(pts)
    verts = vertices_posed.reshape(_V, 3)
    verts_t = jnp.zeros((8, _VPAD), jnp.float32).at[:3, :_V].set(verts.T)

    idx = _nearest_idx(pts8, verts_t)

    ts2 = Ts.reshape(_V, 16)
    pad = jnp.tile(jnp.eye(4, dtype=jnp.float32).reshape(1, 16),
                   (_VPAD2 - _V, 1))
    ts_rows = jnp.concatenate([ts2, pad], axis=0).T
    tinv_rows = _invert_all(ts_rows.reshape(16, _VPAD2 // 128, 128))
    table = jnp.zeros((_VPAD2, 128), jnp.float32)
    table = table.at[:, :16].set(tinv_rows.reshape(16, _VPAD2).T)

    idx3 = idx.reshape(_NW, _NCHUNK, _GCH)
    tg = _sc_gather(table, idx3).reshape(_N, 128)

    outp, outd = _apply_all(pts8, tg)
    can_pts = outp[:, :3].reshape(1, _R, 1, _P, 3)
    can_dirs = outd[:, :3].reshape(1, _R, 1, _P, 3)
    return (can_pts, can_dirs)
